# baseline (speedup 1.0000x reference)
import functools

import jax
import jax.numpy as jnp
from jax import lax
from jax.experimental import pallas as pl
from jax.experimental.pallas import tpu as pltpu
from jax.experimental.pallas import tpu_sc as plsc

N = 10000
E = 320000
D_IN = 128
D_HID = 256
DH = 128

NC = 2
NS = 16
EP_ROWS = 2560
EP = EP_ROWS * 128
DUMMY = N

ROWS_PER_TILE_AGG = EP_ROWS // NS
PHASE_CHUNKS = 4
ROWS_PER_TILE_DEG = EP_ROWS // (NC * NS)
SEED_FULL = 632
SEED_LAST = N - (NS - 1) * SEED_FULL
SEED_LAST_OFF = (NS - 1) * SEED_FULL
DEG_N = NS * SEED_FULL
DEG_PER_TILE = SEED_FULL

ROW_BLK = 2000


def _sc_mesh():
    return plsc.VectorSubcoreMesh(core_axis_name="c", subcore_axis_name="s")


def _deg_body(col_hbm, ones_hbm, zeros_hbm, deg_out, colbuf, ones_v, deg_sp):
    c = lax.axis_index("c")
    s = lax.axis_index("s")
    wid = c * NS + s

    r0 = s * DEG_PER_TILE
    pltpu.sync_copy(zeros_hbm.at[pl.ds(r0, DEG_PER_TILE)],
                    deg_sp.at[pl.ds(r0, DEG_PER_TILE)])
    pltpu.sync_copy(ones_hbm, ones_v)
    pltpu.sync_copy(col_hbm.at[pl.ds(wid * ROWS_PER_TILE_DEG, ROWS_PER_TILE_DEG)],
                    colbuf)
    plsc.subcore_barrier()

    def body(j, carry):
        pltpu.sync_copy(ones_v, deg_sp.at[colbuf.at[j]], add=True)
        return carry

    lax.fori_loop(0, ROWS_PER_TILE_DEG, body, 0)
    plsc.subcore_barrier()

    pltpu.sync_copy(deg_sp.at[pl.ds(r0, DEG_PER_TILE)],
                    deg_out.at[pl.ds(c * DEG_N + r0, DEG_PER_TILE)])


def _deg_call(col2d, ones128, zeros128):
    kern = pl.kernel(
        _deg_body,
        out_type=jax.ShapeDtypeStruct((NC * DEG_N, 128), jnp.float32),
        mesh=_sc_mesh(),
        scratch_types=[
            pltpu.VMEM((ROWS_PER_TILE_DEG, 128), jnp.int32),
            pltpu.VMEM((128, 128), jnp.float32),
            pltpu.VMEM_SHARED((DEG_N, 128), jnp.float32),
        ],
        name="gcn_degree_sc",
    )
    return kern(col2d, ones128, zeros128)


def _agg_body(h_hbm, rc_hbm, out_hbm, idxbuf, bufa, bufb, bufc,
              ga, gb, gc, sa, sb, sc_, agg_sp):
    c = lax.axis_index("c")
    s = lax.axis_index("s")
    base = c * N

    r0 = s * SEED_FULL

    @pl.when(s < NS - 1)
    def _():
        pltpu.sync_copy(h_hbm.at[pl.ds(base + r0, SEED_FULL)],
                        agg_sp.at[pl.ds(r0, SEED_FULL)])

    @pl.when(s == NS - 1)
    def _():
        pltpu.sync_copy(h_hbm.at[pl.ds(base + SEED_LAST_OFF, SEED_LAST)],
                        agg_sp.at[pl.ds(SEED_LAST_OFF, SEED_LAST)])

    e2 = 2 * s * ROWS_PER_TILE_AGG
    plsc.subcore_barrier()

    def phase(p, carry):
        pltpu.sync_copy(rc_hbm.at[c, pl.ds(e2 + p * 2 * PHASE_CHUNKS,
                                           2 * PHASE_CHUNKS)], idxbuf)

        def g(k, buf, sem):
            return pltpu.async_copy(h_hbm.at[idxbuf.at[2 * k]], buf, sem)

        def sc(k, buf, sem):
            return pltpu.async_copy(buf, agg_sp.at[idxbuf.at[2 * k + 1]], sem,
                                    add=True)

        dga = g(0, bufa, ga)
        dgb = g(1, bufb, gb)
        dgc = g(2, bufc, gc)
        dga.wait()
        dsa = sc(0, bufa, sa)
        dgb.wait()
        dsb = sc(1, bufb, sb)
        dsa.wait()
        dga2 = g(3, bufa, ga)
        dgc.wait()
        dsc = sc(2, bufc, sc_)
        dga2.wait()
        dsa2 = sc(3, bufa, sa)
        dsb.wait()
        dsc.wait()
        dsa2.wait()
        return carry

    lax.fori_loop(0, ROWS_PER_TILE_AGG // PHASE_CHUNKS, phase, 0)
    plsc.subcore_barrier()

    @pl.when(s < NS - 1)
    def _():
        pltpu.sync_copy(agg_sp.at[pl.ds(r0, SEED_FULL)],
                        out_hbm.at[pl.ds(base + r0, SEED_FULL)])

    @pl.when(s == NS - 1)
    def _():
        pltpu.sync_copy(agg_sp.at[pl.ds(SEED_LAST_OFF, SEED_LAST)],
                        out_hbm.at[pl.ds(base + SEED_LAST_OFF, SEED_LAST)])


def _agg_call(h_cat, rc):
    kern = pl.kernel(
        _agg_body,
        out_type=jax.ShapeDtypeStruct((NC * N, DH), jnp.float32),
        mesh=_sc_mesh(),
        scratch_types=[
            pltpu.VMEM((2 * PHASE_CHUNKS, 128), jnp.int32),
            pltpu.VMEM((128, DH), jnp.float32),
            pltpu.VMEM((128, DH), jnp.float32),
            pltpu.VMEM((128, DH), jnp.float32),
            pltpu.SemaphoreType.DMA,
            pltpu.SemaphoreType.DMA,
            pltpu.SemaphoreType.DMA,
            pltpu.SemaphoreType.DMA,
            pltpu.SemaphoreType.DMA,
            pltpu.SemaphoreType.DMA,
            pltpu.VMEM_SHARED((N + 8, DH), jnp.float32),
        ],
        name="gcn_aggregate_sc",
    )
    return kern(h_cat, rc)


def _dinv(d0_ref, d1_ref):
    deg = d0_ref[:, 0:1] + d1_ref[:, 0:1] + 1.0
    return lax.rsqrt(deg)


def _mm1_body(x_ref, w_ref, d0_ref, d1_ref, out_ref):
    dinv = _dinv(d0_ref, d1_ref)
    h = jnp.dot(x_ref[:, :], w_ref[:, :],
                preferred_element_type=jnp.float32) * dinv
    out_ref[0, :, :] = h[:, :DH]
    out_ref[1, :, :] = h[:, DH:]


def _mm1_call(x, W1, deg0, deg1):
    grid = N // ROW_BLK
    return pl.pallas_call(
        _mm1_body,
        grid=(grid,),
        in_specs=[
            pl.BlockSpec((ROW_BLK, D_IN), lambda i: (i, 0)),
            pl.BlockSpec((D_IN, D_HID), lambda i: (0, 0)),
            pl.BlockSpec((ROW_BLK, 128), lambda i: (i, 0)),
            pl.BlockSpec((ROW_BLK, 128), lambda i: (i, 0)),
        ],
        out_specs=pl.BlockSpec((2, ROW_BLK, DH), lambda i: (0, i, 0)),
        out_shape=jax.ShapeDtypeStruct((2, N, DH), jnp.float32),
        name="gcn_mm1_tc",
    )(x, W1, deg0, deg1)


def _mm2_body(a0_ref, a1_ref, d0_ref, d1_ref, b1_ref, w_ref, out_ref):
    dinv = _dinv(d0_ref, d1_ref)
    a = jnp.concatenate([a0_ref[0, :, :], a1_ref[0, :, :]], axis=1)
    h = jnp.maximum(a * dinv + b1_ref[0, :][None, :], 0.0)
    h2 = jnp.dot(h, w_ref[:, :], preferred_element_type=jnp.float32) * dinv
    out_ref[0, :, :] = h2[:, :DH]
    out_ref[1, :, :] = h2[:, DH:]


def _mm2_call(agg, deg0, deg1, b1, W2):
    grid = N // ROW_BLK
    agg3 = agg.reshape(2, N, DH)
    return pl.pallas_call(
        _mm2_body,
        grid=(grid,),
        in_specs=[
            pl.BlockSpec((1, ROW_BLK, DH), lambda i: (0, i, 0)),
            pl.BlockSpec((1, ROW_BLK, DH), lambda i: (1, i, 0)),
            pl.BlockSpec((ROW_BLK, 128), lambda i: (i, 0)),
            pl.BlockSpec((ROW_BLK, 128), lambda i: (i, 0)),
            pl.BlockSpec((1, D_HID), lambda i: (0, 0)),
            pl.BlockSpec((D_HID, D_HID), lambda i: (0, 0)),
        ],
        out_specs=pl.BlockSpec((2, ROW_BLK, DH), lambda i: (0, i, 0)),
        out_shape=jax.ShapeDtypeStruct((2, N, DH), jnp.float32),
        name="gcn_mm2_tc",
    )(agg3, agg3, deg0, deg1, b1, W2)


def _final_body(a0_ref, a1_ref, d0_ref, d1_ref, b2_ref, out_ref):
    dinv = _dinv(d0_ref, d1_ref)
    a = jnp.concatenate([a0_ref[0, :, :], a1_ref[0, :, :]], axis=1)
    out_ref[:, :] = a * dinv + b2_ref[0, :][None, :]


def _final_call(agg, deg0, deg1, b2):
    grid = N // ROW_BLK
    agg3 = agg.reshape(2, N, DH)
    return pl.pallas_call(
        _final_body,
        grid=(grid,),
        in_specs=[
            pl.BlockSpec((1, ROW_BLK, DH), lambda i: (0, i, 0)),
            pl.BlockSpec((1, ROW_BLK, DH), lambda i: (1, i, 0)),
            pl.BlockSpec((ROW_BLK, 128), lambda i: (i, 0)),
            pl.BlockSpec((ROW_BLK, 128), lambda i: (i, 0)),
            pl.BlockSpec((1, D_HID), lambda i: (0, 0)),
        ],
        out_specs=pl.BlockSpec((ROW_BLK, D_HID), lambda i: (i, 0)),
        out_shape=jax.ShapeDtypeStruct((N, D_HID), jnp.float32),
        name="gcn_final_tc",
    )(agg3, agg3, deg0, deg1, b2)


def kernel(x, edge_index, W1, b1, W2, b2):
    row = edge_index[0]
    col = edge_index[1]
    pad = EP - E
    row_p = jnp.concatenate([row, jnp.zeros((pad,), jnp.int32)])
    col_p = jnp.concatenate([col, jnp.full((pad,), DUMMY, jnp.int32)])
    col2d = col_p.reshape(EP_ROWS, 128)
    row2d = row_p.reshape(EP_ROWS, 128)
    rowsel = jnp.stack([row2d, row2d + N])
    rc = jnp.stack(
        [rowsel, jnp.broadcast_to(col2d, (NC, EP_ROWS, 128))],
        axis=2).reshape(NC, 2 * EP_ROWS, 128)

    ones128 = jnp.ones((128, 128), jnp.float32)
    zeros128 = jnp.zeros((DEG_N, 128), jnp.float32)
    b1r = b1.reshape(1, D_HID)
    b2r = b2.reshape(1, D_HID)

    degp = _deg_call(col2d, ones128, zeros128)
    deg0 = degp[:N]
    deg1 = degp[DEG_N:DEG_N + N]

    h1 = _mm1_call(x, W1, deg0, deg1).reshape(NC * N, DH)
    agg1 = _agg_call(h1, rc)
    h2 = _mm2_call(agg1, deg0, deg1, b1r, W2).reshape(NC * N, DH)
    agg2 = _agg_call(h2, rc)
    return _final_call(agg2, deg0, deg1, b2r)


# speedup vs baseline: 1.1340x; 1.1340x over previous
import functools

import jax
import jax.numpy as jnp
from jax import lax
from jax.experimental import pallas as pl
from jax.experimental.pallas import tpu as pltpu
from jax.experimental.pallas import tpu_sc as plsc

N = 10000
E = 320000
D_IN = 128
D_HID = 256
DH = 128

NC = 2
NS = 16
EP_ROWS = 2560
EP = EP_ROWS * 128
DUMMY = N

ROWS_PER_TILE_AGG = EP_ROWS // NS
IDX_ROWS = 40
IDX_PHASES = ROWS_PER_TILE_AGG // IDX_ROWS
ROWS_PER_TILE_DEG = EP_ROWS // (NC * NS)
SEED_FULL = 632
SEED_LAST = N - (NS - 1) * SEED_FULL
SEED_LAST_OFF = (NS - 1) * SEED_FULL
DEG_N = NS * SEED_FULL
DEG_PER_TILE = SEED_FULL

ROW_BLK = 2000


def _sc_mesh():
    return plsc.VectorSubcoreMesh(core_axis_name="c", subcore_axis_name="s")


def _deg_body(col_hbm, ones_hbm, zeros_hbm, deg_out, colbuf, ones_v, deg_sp):
    c = lax.axis_index("c")
    s = lax.axis_index("s")
    wid = c * NS + s

    r0 = s * DEG_PER_TILE
    pltpu.sync_copy(zeros_hbm.at[pl.ds(r0, DEG_PER_TILE)],
                    deg_sp.at[pl.ds(r0, DEG_PER_TILE)])
    pltpu.sync_copy(ones_hbm, ones_v)
    pltpu.sync_copy(col_hbm.at[pl.ds(wid * ROWS_PER_TILE_DEG, ROWS_PER_TILE_DEG)],
                    colbuf)
    plsc.subcore_barrier()

    def body(j, carry):
        pltpu.sync_copy(ones_v, deg_sp.at[colbuf.at[j]], add=True)
        return carry

    lax.fori_loop(0, ROWS_PER_TILE_DEG, body, 0)
    plsc.subcore_barrier()

    pltpu.sync_copy(deg_sp.at[pl.ds(r0, DEG_PER_TILE)],
                    deg_out.at[pl.ds(c * DEG_N + r0, DEG_PER_TILE)])


def _deg_call(col2d, ones128, zeros128):
    kern = pl.kernel(
        _deg_body,
        out_type=jax.ShapeDtypeStruct((NC * DEG_N, 128), jnp.float32),
        mesh=_sc_mesh(),
        scratch_types=[
            pltpu.VMEM((ROWS_PER_TILE_DEG, 128), jnp.int32),
            pltpu.VMEM((128, 128), jnp.float32),
            pltpu.VMEM_SHARED((DEG_N, 128), jnp.float32),
        ],
        name="gcn_degree_sc",
    )
    return kern(col2d, ones128, zeros128)


def _agg_body(h_hbm, rowsel_hbm, col_hbm, out_hbm, rowbuf, colbuf, rbuf0,
              rbuf1, gsem0, gsem1, ssem0, ssem1, agg_sp):
    c = lax.axis_index("c")
    s = lax.axis_index("s")
    base = c * N

    r0 = s * SEED_FULL

    @pl.when(s < NS - 1)
    def _():
        pltpu.sync_copy(h_hbm.at[pl.ds(base + r0, SEED_FULL)],
                        agg_sp.at[pl.ds(r0, SEED_FULL)])

    @pl.when(s == NS - 1)
    def _():
        pltpu.sync_copy(h_hbm.at[pl.ds(base + SEED_LAST_OFF, SEED_LAST)],
                        agg_sp.at[pl.ds(SEED_LAST_OFF, SEED_LAST)])

    e0 = s * ROWS_PER_TILE_AGG
    plsc.subcore_barrier()

    def phase(p, carry):
        pltpu.sync_copy(rowsel_hbm.at[c, pl.ds(e0 + p * IDX_ROWS, IDX_ROWS)],
                        rowbuf)
        pltpu.sync_copy(col_hbm.at[pl.ds(e0 + p * IDX_ROWS, IDX_ROWS)], colbuf)
        pltpu.async_copy(h_hbm.at[rowbuf.at[0]], rbuf0, gsem0)

        def pair(q, inner):
            j0 = 2 * q
            j1 = j0 + 1
            pltpu.make_async_copy(h_hbm.at[rowbuf.at[j0]], rbuf0, gsem0).wait()
            g1 = pltpu.async_copy(h_hbm.at[rowbuf.at[j1]], rbuf1, gsem1)
            s0 = pltpu.async_copy(rbuf0, agg_sp.at[colbuf.at[j0]], ssem0,
                                  add=True)
            g1.wait()
            s0.wait()

            @pl.when(q < IDX_ROWS // 2 - 1)
            def _():
                pltpu.async_copy(h_hbm.at[rowbuf.at[j0 + 2]], rbuf0, gsem0)

            s1 = pltpu.async_copy(rbuf1, agg_sp.at[colbuf.at[j1]], ssem1,
                                  add=True)
            s1.wait()
            return inner

        lax.fori_loop(0, IDX_ROWS // 2, pair, 0)
        return carry

    lax.fori_loop(0, IDX_PHASES, phase, 0)
    plsc.subcore_barrier()

    @pl.when(s < NS - 1)
    def _():
        pltpu.sync_copy(agg_sp.at[pl.ds(r0, SEED_FULL)],
                        out_hbm.at[pl.ds(base + r0, SEED_FULL)])

    @pl.when(s == NS - 1)
    def _():
        pltpu.sync_copy(agg_sp.at[pl.ds(SEED_LAST_OFF, SEED_LAST)],
                        out_hbm.at[pl.ds(base + SEED_LAST_OFF, SEED_LAST)])


def _agg_call(h_cat, rowsel, col2d):
    kern = pl.kernel(
        _agg_body,
        out_type=jax.ShapeDtypeStruct((NC * N, DH), jnp.float32),
        mesh=_sc_mesh(),
        scratch_types=[
            pltpu.VMEM((IDX_ROWS, 128), jnp.int32),
            pltpu.VMEM((IDX_ROWS, 128), jnp.int32),
            pltpu.VMEM((128, DH), jnp.float32),
            pltpu.VMEM((128, DH), jnp.float32),
            pltpu.SemaphoreType.DMA,
            pltpu.SemaphoreType.DMA,
            pltpu.SemaphoreType.DMA,
            pltpu.SemaphoreType.DMA,
            pltpu.VMEM_SHARED((N + 8, DH), jnp.float32),
        ],
        name="gcn_aggregate_sc",
    )
    return kern(h_cat, rowsel, col2d)


def _dinv(d0_ref, d1_ref):
    deg = d0_ref[:, 0:1] + d1_ref[:, 0:1] + 1.0
    return lax.rsqrt(deg)


def _mm1_body(x_ref, w_ref, d0_ref, d1_ref, out_ref):
    dinv = _dinv(d0_ref, d1_ref)
    h = jnp.dot(x_ref[:, :], w_ref[:, :],
                preferred_element_type=jnp.float32) * dinv
    out_ref[0, :, :] = h[:, :DH]
    out_ref[1, :, :] = h[:, DH:]


def _mm1_call(x, W1, deg0, deg1):
    grid = N // ROW_BLK
    return pl.pallas_call(
        _mm1_body,
        grid=(grid,),
        in_specs=[
            pl.BlockSpec((ROW_BLK, D_IN), lambda i: (i, 0)),
            pl.BlockSpec((D_IN, D_HID), lambda i: (0, 0)),
            pl.BlockSpec((ROW_BLK, 128), lambda i: (i, 0)),
            pl.BlockSpec((ROW_BLK, 128), lambda i: (i, 0)),
        ],
        out_specs=pl.BlockSpec((2, ROW_BLK, DH), lambda i: (0, i, 0)),
        out_shape=jax.ShapeDtypeStruct((2, N, DH), jnp.float32),
        name="gcn_mm1_tc",
    )(x, W1, deg0, deg1)


def _mm2_body(a0_ref, a1_ref, d0_ref, d1_ref, b1_ref, w_ref, out_ref):
    dinv = _dinv(d0_ref, d1_ref)
    a = jnp.concatenate([a0_ref[0, :, :], a1_ref[0, :, :]], axis=1)
    h = jnp.maximum(a * dinv + b1_ref[0, :][None, :], 0.0)
    h2 = jnp.dot(h, w_ref[:, :], preferred_element_type=jnp.float32) * dinv
    out_ref[0, :, :] = h2[:, :DH]
    out_ref[1, :, :] = h2[:, DH:]


def _mm2_call(agg, deg0, deg1, b1, W2):
    grid = N // ROW_BLK
    agg3 = agg.reshape(2, N, DH)
    return pl.pallas_call(
        _mm2_body,
        grid=(grid,),
        in_specs=[
            pl.BlockSpec((1, ROW_BLK, DH), lambda i: (0, i, 0)),
            pl.BlockSpec((1, ROW_BLK, DH), lambda i: (1, i, 0)),
            pl.BlockSpec((ROW_BLK, 128), lambda i: (i, 0)),
            pl.BlockSpec((ROW_BLK, 128), lambda i: (i, 0)),
            pl.BlockSpec((1, D_HID), lambda i: (0, 0)),
            pl.BlockSpec((D_HID, D_HID), lambda i: (0, 0)),
        ],
        out_specs=pl.BlockSpec((2, ROW_BLK, DH), lambda i: (0, i, 0)),
        out_shape=jax.ShapeDtypeStruct((2, N, DH), jnp.float32),
        name="gcn_mm2_tc",
    )(agg3, agg3, deg0, deg1, b1, W2)


def _final_body(a0_ref, a1_ref, d0_ref, d1_ref, b2_ref, out_ref):
    dinv = _dinv(d0_ref, d1_ref)
    a = jnp.concatenate([a0_ref[0, :, :], a1_ref[0, :, :]], axis=1)
    out_ref[:, :] = a * dinv + b2_ref[0, :][None, :]


def _final_call(agg, deg0, deg1, b2):
    grid = N // ROW_BLK
    agg3 = agg.reshape(2, N, DH)
    return pl.pallas_call(
        _final_body,
        grid=(grid,),
        in_specs=[
            pl.BlockSpec((1, ROW_BLK, DH), lambda i: (0, i, 0)),
            pl.BlockSpec((1, ROW_BLK, DH), lambda i: (1, i, 0)),
            pl.BlockSpec((ROW_BLK, 128), lambda i: (i, 0)),
            pl.BlockSpec((ROW_BLK, 128), lambda i: (i, 0)),
            pl.BlockSpec((1, D_HID), lambda i: (0, 0)),
        ],
        out_specs=pl.BlockSpec((ROW_BLK, D_HID), lambda i: (i, 0)),
        out_shape=jax.ShapeDtypeStruct((N, D_HID), jnp.float32),
        name="gcn_final_tc",
    )(agg3, agg3, deg0, deg1, b2)


def kernel(x, edge_index, W1, b1, W2, b2):
    row = edge_index[0]
    col = edge_index[1]
    pad = EP - E
    row_p = jnp.concatenate([row, jnp.zeros((pad,), jnp.int32)])
    col_p = jnp.concatenate([col, jnp.full((pad,), DUMMY, jnp.int32)])
    col2d = col_p.reshape(EP_ROWS, 128)
    row2d = row_p.reshape(EP_ROWS, 128)
    rowsel = jnp.stack([row2d, row2d + N])

    ones128 = jnp.ones((128, 128), jnp.float32)
    zeros128 = jnp.zeros((DEG_N, 128), jnp.float32)
    b1r = b1.reshape(1, D_HID)
    b2r = b2.reshape(1, D_HID)

    degp = _deg_call(col2d, ones128, zeros128)
    deg0 = degp[:N]
    deg1 = degp[DEG_N:DEG_N + N]

    h1 = _mm1_call(x, W1, deg0, deg1).reshape(NC * N, DH)
    agg1 = _agg_call(h1, rowsel, col2d)
    h2 = _mm2_call(agg1, deg0, deg1, b1r, W2).reshape(NC * N, DH)
    agg2 = _agg_call(h2, rowsel, col2d)
    return _final_call(agg2, deg0, deg1, b2r)
